# baseline (speedup 1.0000x reference)
import functools

import jax
import jax.numpy as jnp
from jax import lax
from jax.experimental import pallas as pl
from jax.experimental.pallas import tpu as pltpu
from jax.experimental.pallas import tpu_sc as plsc

B = 4096
F = 26
NUM_ROWS_PER_FIELD = 100000
NW = 32
BPW = B // NW
NPW = BPW * F
LANES = 16
CPW = BPW // LANES


def _make_kernel():
    mesh = plsc.VectorSubcoreMesh(core_axis_name="c", subcore_axis_name="s")

    @functools.partial(
        pl.kernel,
        out_type=jax.ShapeDtypeStruct((B,), jnp.float32),
        mesh=mesh,
        scratch_types=[
            pltpu.VMEM((NPW,), jnp.int32),
            pltpu.VMEM((NPW,), jnp.float32),
            pltpu.VMEM((LANES,), jnp.float32),
            pltpu.VMEM((BPW,), jnp.float32),
            pltpu.SemaphoreType.DMA,
            pltpu.SemaphoreType.DMA,
        ],
    )
    def k(xt_hbm, table_hbm, bias_hbm, out_hbm,
          idx_v, vals_v, bias_v, out_v, sem, bsem):
        wid = lax.axis_index("s") * 2 + lax.axis_index("c")
        base = wid * NPW

        bias_cp = pltpu.async_copy(bias_hbm, bias_v, bsem)
        pltpu.sync_copy(xt_hbm.at[pl.ds(base, NPW)], idx_v)

        for f in range(F):
            off = jnp.int32(f * NUM_ROWS_PER_FIELD)
            for c in range(CPW):
                sl = pl.ds(f * BPW + c * LANES, LANES)
                idx_v[sl] = idx_v[sl] + off

        pltpu.async_copy(table_hbm.at[idx_v], vals_v, sem).wait()
        bias_cp.wait()

        bias16 = bias_v[...]
        accs = [bias16] * CPW
        for f in range(F):
            for c in range(CPW):
                accs[c] = accs[c] + vals_v[pl.ds(f * BPW + c * LANES, LANES)]
        for c in range(CPW):
            out_v[pl.ds(c * LANES, LANES)] = accs[c]

        pltpu.sync_copy(out_v, out_hbm.at[pl.ds(wid * BPW, BPW)])

    return k


_sc_kernel = _make_kernel()


def kernel(x, table, bias):
    xt = x.T.reshape(F, NW, BPW).transpose(1, 0, 2).reshape(-1)
    tablef = table.reshape(-1)
    bias16 = jnp.broadcast_to(bias.astype(jnp.float32), (LANES,))
    out = _sc_kernel(xt, tablef, bias16)
    return out.reshape(B, 1)
